# baseline (speedup 1.0000x reference)
import math

import numpy as np
import jax
import jax.numpy as jnp
from jax import lax
from jax.experimental import pallas as pl
from jax.experimental.pallas import tpu as pltpu
from jax.experimental.pallas import tpu_sc as plsc

_NUM_EMB = 1000000
_DIM = 64
_BATCH = 4096
_SEQ = 200
_LANES = 16

_NC, _NS = 2, 16
_NW = _NC * _NS
_BBLK = _BATCH // _NW
_NG = _BBLK // _LANES

_NPAIR = _NUM_EMB // 2
_RBLK = 256
_NBLK = _NUM_EMB // _RBLK
_KMAX = (_NBLK - 1) // _NW + 1


def _pos_encoding():
    pos = np.arange(_SEQ, dtype=np.float32)[:, None]
    div = np.exp(np.arange(0, _DIM, 2, dtype=np.float32)
                 * -(math.log(10000.0) / _DIM))
    pe = np.zeros((_SEQ, _DIM), dtype=np.float32)
    pe[:, 0::2] = np.sin(pos * div)
    pe[:, 1::2] = np.cos(pos * div)
    return pe.reshape(-1)


_PE = _pos_encoding()


def _transpose_block(tin, tout, rowqs, iota, nrows):
    @plsc.parallel_loop(0, nrows, unroll=16)
    def prow(p):
        for h in range(2):
            col = iota * 0 + (2 * p + h)
            for q in range(_DIM // _LANES):
                v = plsc.load_gather(tin, [rowqs[q], col])
                tout[p, pl.ds(h * _DIM + q * _LANES, _LANES)] = v


def _pack_body(tablet_hbm, tailp_hbm, pairs_hbm,
               tin0, tin1, tout0, tout1, ttail_v, ism0, ism1, osm0, osm1):
    wid = lax.axis_index("s") * _NC + lax.axis_index("c")
    iota = lax.iota(jnp.int32, _LANES)
    rowqs = [q * _LANES + iota for q in range(_DIM // _LANES)]
    tins = (tin0, tin1)
    touts = (tout0, tout1)
    isms = (ism0, ism1)
    osms = (osm0, osm1)

    def start_in(blk, b):
        pltpu.async_copy(tablet_hbm.at[:, pl.ds(blk * _RBLK, _RBLK)],
                         tins[b], isms[b])

    start_in(wid, 0)

    def step2(j, carry):
        for b in range(2):
            k = 2 * j + b
            blk = wid + k * _NW

            @pl.when(blk < _NBLK)
            def _():
                nblk = blk + _NW

                @pl.when(nblk < _NBLK)
                def _():
                    start_in(nblk, 1 - b)

                pltpu.make_async_copy(
                    tablet_hbm.at[:, pl.ds(blk * _RBLK, _RBLK)],
                    tins[b], isms[b]).wait()

                @pl.when(k >= 2)
                def _():
                    pltpu.make_async_copy(
                        touts[b], pairs_hbm.at[pl.ds(0, _RBLK // 2)],
                        osms[b]).wait()

                _transpose_block(tins[b], touts[b], rowqs, iota, _RBLK // 2)
                pltpu.async_copy(touts[b],
                                 pairs_hbm.at[pl.ds(blk * (_RBLK // 2),
                                                    _RBLK // 2)],
                                 osms[b])
        return carry

    lax.fori_loop(0, (_KMAX + 1) // 2, step2, 0)

    for b in range(2):
        pltpu.make_async_copy(touts[b], pairs_hbm.at[pl.ds(0, _RBLK // 2)],
                              osms[b]).wait()

    @pl.when(wid == 0)
    def _():
        pltpu.sync_copy(tailp_hbm, ttail_v)
        pltpu.sync_copy(ttail_v,
                        pairs_hbm.at[pl.ds(_NBLK * (_RBLK // 2), 32)])


def _emb_body(xt_hbm, pe_hbm, pairs_hbm, out_hbm,
              xall_v, pe_v, idxp0, idxp1, gath0, gath1, ost0, ost1,
              gsm0, gsm1, osm0, osm1):
    wid = lax.axis_index("s") * _NC + lax.axis_index("c")
    b0 = wid * _BBLK
    pltpu.sync_copy(pe_hbm, pe_v)
    pltpu.sync_copy(xt_hbm.at[:, pl.ds(b0, _BBLK)], xall_v)
    iota = lax.iota(jnp.int32, _LANES)
    rowids = [g * _LANES + iota for g in range(_NG)]
    idxps = (idxp0, idxp1)
    gaths = (gath0, gath1)
    osts = (ost0, ost1)
    gsms = (gsm0, gsm1)
    osms = (osm0, osm1)

    def start_gather(l, b):
        for g in range(_NG):
            sl = pl.ds(g * _LANES, _LANES)
            idxps[b][sl] = xall_v[l, sl] >> 1
        pltpu.async_copy(pairs_hbm.at[idxps[b]], gaths[b], gsms[b])

    start_gather(0, 0)

    def step2(j, carry):
        for b in range(2):
            l = 2 * j + b

            @pl.when(l + 1 < _SEQ)
            def _():
                start_gather(l + 1, 1 - b)

            pltpu.make_async_copy(pairs_hbm.at[idxps[b]], gaths[b],
                                  gsms[b]).wait()

            @pl.when(l >= 2)
            def _():
                pltpu.make_async_copy(
                    osts[b], out_hbm.at[0, :, pl.ds(b0, _BBLK)],
                    osms[b]).wait()

            pars = []
            for g in range(_NG):
                xv = xall_v[l, pl.ds(g * _LANES, _LANES)]
                pars.append((xv & 1) << 6)
            pe_base = iota * 0 + l * _DIM

            @plsc.parallel_loop(0, _DIM, unroll=16)
            def col(d):
                pv = plsc.load_gather(pe_v, [pe_base + d])
                for g in range(_NG):
                    cv = plsc.load_gather(gaths[b], [rowids[g], pars[g] + d])
                    osts[b][d, pl.ds(g * _LANES, _LANES)] = cv + pv
            pltpu.async_copy(osts[b], out_hbm.at[l, :, pl.ds(b0, _BBLK)],
                             osms[b])
        return carry

    lax.fori_loop(0, _SEQ // 2, step2, 0)
    for b in range(2):
        pltpu.make_async_copy(osts[b], out_hbm.at[0, :, pl.ds(b0, _BBLK)],
                              osms[b]).wait()


@jax.jit
def _run(x, pe, table):
    mesh = plsc.VectorSubcoreMesh(core_axis_name="c", subcore_axis_name="s")
    pack = pl.kernel(
        _pack_body,
        out_type=jax.ShapeDtypeStruct((_NPAIR, 2 * _DIM), jnp.float32),
        mesh=mesh,
        scratch_types=[
            pltpu.VMEM((_DIM, _RBLK), jnp.float32),
            pltpu.VMEM((_DIM, _RBLK), jnp.float32),
            pltpu.VMEM((_RBLK // 2, 2 * _DIM), jnp.float32),
            pltpu.VMEM((_RBLK // 2, 2 * _DIM), jnp.float32),
            pltpu.VMEM((32, 2 * _DIM), jnp.float32),
            pltpu.SemaphoreType.DMA,
            pltpu.SemaphoreType.DMA,
            pltpu.SemaphoreType.DMA,
            pltpu.SemaphoreType.DMA,
        ],
        compiler_params=pltpu.CompilerParams(needs_layout_passes=False),
    )
    emb = pl.kernel(
        _emb_body,
        out_type=jax.ShapeDtypeStruct((_SEQ, _DIM, _BATCH), jnp.float32),
        mesh=mesh,
        scratch_types=[
            pltpu.VMEM((_SEQ, _BBLK), jnp.int32),
            pltpu.VMEM((_SEQ * _DIM,), jnp.float32),
            pltpu.VMEM((_BBLK,), jnp.int32),
            pltpu.VMEM((_BBLK,), jnp.int32),
            pltpu.VMEM((_BBLK, 2 * _DIM), jnp.float32),
            pltpu.VMEM((_BBLK, 2 * _DIM), jnp.float32),
            pltpu.VMEM((_DIM, _BBLK), jnp.float32),
            pltpu.VMEM((_DIM, _BBLK), jnp.float32),
            pltpu.SemaphoreType.DMA,
            pltpu.SemaphoreType.DMA,
            pltpu.SemaphoreType.DMA,
            pltpu.SemaphoreType.DMA,
        ],
        compiler_params=pltpu.CompilerParams(needs_layout_passes=False),
    )
    tailp = table[_NBLK * _RBLK:].reshape(32, 2 * _DIM)
    pairs = pack(table.T, tailp)
    out2 = emb(x.T, pe, pairs)
    return jnp.transpose(out2, (2, 0, 1))


def kernel(x, table):
    return _run(x, _PE, table)


# speedup vs baseline: 1.2126x; 1.2126x over previous
import math

import numpy as np
import jax
import jax.numpy as jnp
from jax import lax
from jax.experimental import pallas as pl
from jax.experimental.pallas import tpu as pltpu
from jax.experimental.pallas import tpu_sc as plsc

_NUM_EMB = 1000000
_DIM = 64
_BATCH = 4096
_SEQ = 200
_LANES = 16

_NC, _NS = 2, 16
_NW = _NC * _NS
_SEQ_PER_W = _BATCH // _NW
_CHUNK_SEQ = 2
_ROWS = _CHUNK_SEQ * _SEQ
_N_CHUNKS = _SEQ_PER_W // _CHUNK_SEQ


def _pos_encoding():
    pos = np.arange(_SEQ, dtype=np.float32)[:, None]
    div = np.exp(np.arange(0, _DIM, 2, dtype=np.float32)
                 * -(math.log(10000.0) / _DIM))
    pe = np.zeros((_SEQ, _DIM), dtype=np.float32)
    pe[:, 0::2] = np.sin(pos * div)
    pe[:, 1::2] = np.cos(pos * div)
    return np.tile(pe, (_CHUNK_SEQ, 1))


_PE = _pos_encoding()


def _body(x_hbm, pe_hbm, table_hbm, out_hbm,
          idx0, idx1, rows0, rows1, pe_v, gsm0, gsm1, osm0, osm1):
    wid = lax.axis_index("s") * _NC + lax.axis_index("c")
    seq0 = wid * _SEQ_PER_W
    pltpu.sync_copy(pe_hbm, pe_v)
    idxs = (idx0, idx1)
    rows = (rows0, rows1)
    gsms = (gsm0, gsm1)
    osms = (osm0, osm1)

    def start_gather(i, b):
        row0 = (seq0 + i * _CHUNK_SEQ) * _SEQ
        pltpu.sync_copy(x_hbm.at[pl.ds(row0, _ROWS)], idxs[b])
        pltpu.async_copy(table_hbm.at[idxs[b]], rows[b], gsms[b])

    start_gather(0, 0)

    def step2(j, carry):
        for b in range(2):
            i = 2 * j + b
            s0 = seq0 + i * _CHUNK_SEQ

            @pl.when(i + 1 < _N_CHUNKS)
            def _():
                start_gather(i + 1, 1 - b)

            pltpu.make_async_copy(table_hbm.at[idxs[b]], rows[b],
                                  gsms[b]).wait()

            @pl.when(i >= 2)
            def _():
                for h in range(_CHUNK_SEQ):
                    pltpu.make_async_copy(
                        rows[b].at[pl.ds(h * _SEQ, _SEQ)],
                        out_hbm.at[s0 + h], osms[b]).wait()

            @plsc.parallel_loop(0, _ROWS, unroll=16)
            def add_row(r):
                for q in range(_DIM // _LANES):
                    sl = pl.ds(q * _LANES, _LANES)
                    rows[b][r, sl] = rows[b][r, sl] + pe_v[r, sl]

            for h in range(_CHUNK_SEQ):
                pltpu.async_copy(rows[b].at[pl.ds(h * _SEQ, _SEQ)],
                                 out_hbm.at[s0 + h], osms[b])
        return carry

    lax.fori_loop(0, _N_CHUNKS // 2, step2, 0)
    for b in range(2):
        for h in range(_CHUNK_SEQ):
            pltpu.make_async_copy(rows[b].at[pl.ds(h * _SEQ, _SEQ)],
                                  out_hbm.at[h], osms[b]).wait()


@jax.jit
def _run(x_flat, pe, table):
    mesh = plsc.VectorSubcoreMesh(core_axis_name="c", subcore_axis_name="s")
    f = pl.kernel(
        _body,
        out_type=jax.ShapeDtypeStruct((_BATCH, _SEQ, _DIM), jnp.float32),
        mesh=mesh,
        scratch_types=[
            pltpu.VMEM((_ROWS,), jnp.int32),
            pltpu.VMEM((_ROWS,), jnp.int32),
            pltpu.VMEM((_ROWS, _DIM), jnp.float32),
            pltpu.VMEM((_ROWS, _DIM), jnp.float32),
            pltpu.VMEM((_ROWS, _DIM), jnp.float32),
            pltpu.SemaphoreType.DMA,
            pltpu.SemaphoreType.DMA,
            pltpu.SemaphoreType.DMA,
            pltpu.SemaphoreType.DMA,
        ],
        compiler_params=pltpu.CompilerParams(use_tc_tiling_on_sc=False),
    )
    return f(x_flat, pe, table)


def kernel(x, table):
    return _run(x.reshape(-1), _PE, table)
